# Initial kernel scaffold; baseline (speedup 1.0000x reference)
import jax
import jax.numpy as jnp
from jax.experimental import pallas as pl


def kernel(features, membership_mask, block0_src, block0_dst, block1_src, block1_dst, output_nodes_indices, W_pin, b_pin, W_pout, b_pout, Ws1, Wn1, b1, Ws2, Wn2, b2, Wc, bc):
    raise NotImplementedError("write your pallas kernel here")


# speedup vs baseline: 3.7416x; 3.7416x over previous
import functools

import jax
import jax.numpy as jnp
from jax import lax
from jax.experimental import pallas as pl
from jax.experimental.pallas import tpu as pltpu
from jax.experimental.pallas import tpu_sc as plsc

N0, N1, N2 = 10000, 5000, 2000
E0, E1 = 160000, 64000
IN_DIM, PROMPT, HID, OUT = 128, 64, 128, 64

NC, NS = 2, 16
NW = NC * NS
CHUNK = 128


def _make_seg_sum(n_chunks: int, npad: int):
  e_pw = n_chunks * CHUNK
  rows_ps = npad // NS
  assert npad % NS == 0

  mesh = plsc.VectorSubcoreMesh(core_axis_name="c", subcore_axis_name="s")

  @functools.partial(
      pl.kernel,
      mesh=mesh,
      compiler_params=pltpu.CompilerParams(needs_layout_passes=False),
      out_type=[
          jax.ShapeDtypeStruct((NC, npad, HID), jnp.float32),
          jax.ShapeDtypeStruct((NC, npad), jnp.float32),
      ],
      scratch_types=[
          pltpu.VMEM((CHUNK,), jnp.int32),
          pltpu.VMEM((CHUNK,), jnp.int32),
          pltpu.VMEM((CHUNK, HID), jnp.float32),
          pltpu.VMEM((npad,), jnp.float32),
          pltpu.VMEM((NS, rows_ps), jnp.float32),
          pltpu.VMEM((rows_ps,), jnp.float32),
          pltpu.VMEM_SHARED((npad, HID), jnp.float32),
          pltpu.VMEM_SHARED((NS, npad), jnp.float32),
          pltpu.SemaphoreType.DMA,
      ],
  )
  def seg_sum(table, src, dst, zrows, zcnt, acc_out, cnt_out,
              src_v, dst_v, rows_v, cnt_loc, red_v, cnt_red, acc_sh,
              cnt_stage, sem):
    cid = lax.axis_index("c")
    sid = lax.axis_index("s")
    wid = sid * NC + cid

    r0 = sid * rows_ps
    pltpu.sync_copy(zrows, acc_sh.at[pl.ds(r0, rows_ps)])
    pltpu.sync_copy(zcnt, cnt_loc)
    plsc.subcore_barrier()

    base = wid * e_pw
    ones16 = jnp.ones((16,), jnp.float32)

    def body(i, carry):
      off = base + i * CHUNK
      pltpu.sync_copy(src.at[pl.ds(off, CHUNK)], src_v)
      pltpu.sync_copy(dst.at[pl.ds(off, CHUNK)], dst_v)
      pltpu.async_copy(table.at[src_v], rows_v, sem).wait()
      pltpu.sync_copy(rows_v, acc_sh.at[dst_v], add=True)
      for j in range(CHUNK // 16):
        idx = dst_v[pl.ds(j * 16, 16)]
        plsc.addupdate_scatter(cnt_loc, [idx], ones16)
      return carry

    lax.fori_loop(0, n_chunks, body, 0)
    pltpu.sync_copy(cnt_loc, cnt_stage.at[sid])
    plsc.subcore_barrier()

    pltpu.sync_copy(cnt_stage.at[:, pl.ds(r0, rows_ps)], red_v)

    def red_body(j, carry):
      s = red_v[0, pl.ds(j * 16, 16)]
      for t in range(1, NS):
        s = s + red_v[t, pl.ds(j * 16, 16)]
      cnt_red[pl.ds(j * 16, 16)] = s
      return carry

    lax.fori_loop(0, rows_ps // 16, red_body, 0)

    pltpu.sync_copy(acc_sh.at[pl.ds(r0, rows_ps)],
                    acc_out.at[cid].at[pl.ds(r0, rows_ps)])
    pltpu.sync_copy(cnt_red, cnt_out.at[cid].at[pl.ds(r0, rows_ps)])

  return seg_sum


N1_PAD = 6144
N2_PAD = 2048
L1_CHUNKS = 40
L2_CHUNKS = 16

_seg_sum_l1 = _make_seg_sum(L1_CHUNKS, N1_PAD)
_seg_sum_l2 = _make_seg_sum(L2_CHUNKS, N2_PAD)


BLK = 1000


def _prompt_body(x_ref, m_ref, wpin_ref, bpin_ref, wpout_ref, bpout_ref,
                 wn1_ref, ws1_ref, b1_ref, g0_ref, s1_ref):
  x = x_ref[...]
  pin = jnp.maximum(
      jnp.dot(x, wpin_ref[...], preferred_element_type=jnp.float32)
      + bpin_ref[...], 0.0)
  pout = jnp.maximum(
      jnp.dot(x, wpout_ref[...], preferred_element_type=jnp.float32)
      + bpout_ref[...], 0.0)
  s = jnp.where(m_ref[...] > 0, pin, pout)
  g0_ref[...] = (
      jnp.dot(x, wn1_ref[0:IN_DIM], preferred_element_type=jnp.float32)
      + jnp.dot(s, wn1_ref[IN_DIM:IN_DIM + PROMPT],
                preferred_element_type=jnp.float32))
  s1_ref[...] = (
      jnp.dot(x, ws1_ref[0:IN_DIM], preferred_element_type=jnp.float32)
      + jnp.dot(s, ws1_ref[IN_DIM:IN_DIM + PROMPT],
                preferred_element_type=jnp.float32)
      + b1_ref[...])


def _layer1_body(acc_ref, cnt_ref, s1_ref, wn2_ref, ws2_ref, b2_ref,
                 g1_ref, s2_ref):
  a = acc_ref[0] + acc_ref[1]
  d = cnt_ref[0] + cnt_ref[1]
  h1 = jnp.maximum(s1_ref[...] + a / jnp.maximum(d, 1.0), 0.0)
  g1_ref[...] = jnp.dot(h1, wn2_ref[...], preferred_element_type=jnp.float32)
  s2_ref[...] = (jnp.dot(h1, ws2_ref[...], preferred_element_type=jnp.float32)
                 + b2_ref[...])


def _layer2_body(acc_ref, cnt_ref, s2_ref, wc_ref, bc_ref, out_ref):
  a = acc_ref[0] + acc_ref[1]
  d = cnt_ref[0] + cnt_ref[1]
  h2 = s2_ref[...] + a / jnp.maximum(d, 1.0)
  out_ref[...] = (jnp.dot(h2, wc_ref[...], preferred_element_type=jnp.float32)
                  + bc_ref[...])


def _full(shape):
  return pl.BlockSpec(shape, lambda i: (0,) * len(shape))


def kernel(features, membership_mask, block0_src, block0_dst, block1_src,
           block1_dst, output_nodes_indices, W_pin, b_pin, W_pout, b_pout,
           Ws1, Wn1, b1, Ws2, Wn2, b2, Wc, bc):
  del output_nodes_indices
  f32 = jnp.float32
  mask_i = membership_mask.astype(jnp.int32).reshape(N0, 1)

  g0, s1 = pl.pallas_call(
      _prompt_body,
      grid=(N0 // BLK,),
      in_specs=[
          pl.BlockSpec((BLK, IN_DIM), lambda i: (i, 0)),
          pl.BlockSpec((BLK, 1), lambda i: (i, 0)),
          _full((IN_DIM, PROMPT)),
          _full((1, PROMPT)),
          _full((IN_DIM, PROMPT)),
          _full((1, PROMPT)),
          _full((IN_DIM + PROMPT, HID)),
          _full((IN_DIM + PROMPT, HID)),
          _full((1, HID)),
      ],
      out_specs=[
          pl.BlockSpec((BLK, HID), lambda i: (i, 0)),
          pl.BlockSpec((BLK, HID), lambda i: (i, 0)),
      ],
      out_shape=[
          jax.ShapeDtypeStruct((N0, HID), f32),
          jax.ShapeDtypeStruct((N0, HID), f32),
      ],
  )(features, mask_i, W_pin, b_pin.reshape(1, PROMPT), W_pout,
    b_pout.reshape(1, PROMPT), Wn1, Ws1, b1.reshape(1, HID))

  e0_pad = NW * L1_CHUNKS * CHUNK
  src0 = jnp.concatenate([block0_src, jnp.zeros((e0_pad - E0,), jnp.int32)])
  dst0 = jnp.concatenate([block0_dst, jnp.full((e0_pad - E0,), N1, jnp.int32)])
  zrows1 = jnp.zeros((N1_PAD // NS, HID), f32)
  zcnt1 = jnp.zeros((N1_PAD,), f32)
  acc1, cnt1 = _seg_sum_l1(g0, src0, dst0, zrows1, zcnt1)
  cnt1 = cnt1.reshape(NC, N1_PAD, 1)

  g1, s2 = pl.pallas_call(
      _layer1_body,
      grid=(N1 // BLK,),
      in_specs=[
          pl.BlockSpec((NC, BLK, HID), lambda i: (0, i, 0)),
          pl.BlockSpec((NC, BLK, 1), lambda i: (0, i, 0)),
          pl.BlockSpec((BLK, HID), lambda i: (i, 0)),
          _full((HID, HID)),
          _full((HID, HID)),
          _full((1, HID)),
      ],
      out_specs=[
          pl.BlockSpec((BLK, HID), lambda i: (i, 0)),
          pl.BlockSpec((BLK, HID), lambda i: (i, 0)),
      ],
      out_shape=[
          jax.ShapeDtypeStruct((N1, HID), f32),
          jax.ShapeDtypeStruct((N1, HID), f32),
      ],
  )(acc1, cnt1, s1, Wn2, Ws2, b2.reshape(1, HID))

  e1_pad = NW * L2_CHUNKS * CHUNK
  src1 = jnp.concatenate([block1_src, jnp.zeros((e1_pad - E1,), jnp.int32)])
  dst1 = jnp.concatenate([block1_dst, jnp.full((e1_pad - E1,), N2, jnp.int32)])
  zrows2 = jnp.zeros((N2_PAD // NS, HID), f32)
  zcnt2 = jnp.zeros((N2_PAD,), f32)
  acc2, cnt2 = _seg_sum_l2(g1, src1, dst1, zrows2, zcnt2)
  cnt2 = cnt2.reshape(NC, N2_PAD, 1)

  logits = pl.pallas_call(
      _layer2_body,
      grid=(N2 // BLK,),
      in_specs=[
          pl.BlockSpec((NC, BLK, HID), lambda i: (0, i, 0)),
          pl.BlockSpec((NC, BLK, 1), lambda i: (0, i, 0)),
          pl.BlockSpec((BLK, HID), lambda i: (i, 0)),
          _full((HID, OUT)),
          _full((1, OUT)),
      ],
      out_specs=pl.BlockSpec((BLK, OUT), lambda i: (i, 0)),
      out_shape=jax.ShapeDtypeStruct((N2, OUT), f32),
  )(acc2, cnt2, s2, Wc, bc.reshape(1, OUT))

  return logits


# speedup vs baseline: 4.1966x; 1.1216x over previous
import functools

import jax
import jax.numpy as jnp
from jax import lax
from jax.experimental import pallas as pl
from jax.experimental.pallas import tpu as pltpu
from jax.experimental.pallas import tpu_sc as plsc

N0, N1, N2 = 10000, 5000, 2000
E0, E1 = 160000, 64000
IN_DIM, PROMPT, HID, OUT = 128, 64, 128, 64

NC, NS = 2, 16
NW = NC * NS
CHUNK = 128
NBUF = 2


def _make_seg_sum(n_chunks: int, npad: int):
  e_pw = n_chunks * CHUNK
  rows_ps = npad // NS
  assert npad % NS == 0

  mesh = plsc.VectorSubcoreMesh(core_axis_name="c", subcore_axis_name="s")

  @functools.partial(
      pl.kernel,
      mesh=mesh,
      compiler_params=pltpu.CompilerParams(needs_layout_passes=False),
      out_type=[
          jax.ShapeDtypeStruct((NC, npad, HID), jnp.float32),
          jax.ShapeDtypeStruct((NC, npad), jnp.float32),
      ],
      scratch_types=[
          pltpu.VMEM((NBUF, CHUNK), jnp.int32),
          pltpu.VMEM((NBUF, CHUNK), jnp.int32),
          [pltpu.VMEM((CHUNK, HID), jnp.float32)] * NBUF,
          pltpu.VMEM((npad,), jnp.float32),
          pltpu.VMEM((NS, rows_ps), jnp.float32),
          pltpu.VMEM((rows_ps,), jnp.float32),
          pltpu.VMEM_SHARED((npad, HID), jnp.float32),
          pltpu.VMEM_SHARED((NS, npad), jnp.float32),
          [pltpu.SemaphoreType.DMA] * NBUF,
      ],
  )
  def seg_sum(table, src, dst, zrows, zcnt, acc_out, cnt_out,
              src_v, dst_v, rows_v, cnt_loc, red_v, cnt_red, acc_sh,
              cnt_stage, sems):
    cid = lax.axis_index("c")
    sid = lax.axis_index("s")
    wid = sid * NC + cid

    r0 = sid * rows_ps
    pltpu.sync_copy(zrows, acc_sh.at[pl.ds(r0, rows_ps)])
    pltpu.sync_copy(zcnt, cnt_loc)
    plsc.subcore_barrier()

    base = wid * e_pw
    ones16 = jnp.ones((16,), jnp.float32)

    def body(g, carry):
      i0 = base + g * (NBUF * CHUNK)
      handles = []
      for b in range(NBUF):
        off = i0 + b * CHUNK
        pltpu.sync_copy(src.at[pl.ds(off, CHUNK)], src_v.at[b])
        pltpu.sync_copy(dst.at[pl.ds(off, CHUNK)], dst_v.at[b])
        handles.append(pltpu.async_copy(table.at[src_v.at[b]], rows_v[b],
                                        sems[b]))
      for b in range(NBUF):
        handles[b].wait()
        pltpu.sync_copy(rows_v[b], acc_sh.at[dst_v.at[b]], add=True)
        for j in range(CHUNK // 16):
          idx = dst_v[b, pl.ds(j * 16, 16)]
          plsc.addupdate_scatter(cnt_loc, [idx], ones16)
      return carry

    lax.fori_loop(0, n_chunks // NBUF, body, 0)
    pltpu.sync_copy(cnt_loc, cnt_stage.at[sid])
    plsc.subcore_barrier()

    pltpu.sync_copy(cnt_stage.at[:, pl.ds(r0, rows_ps)], red_v)

    def red_body(j, carry):
      s = red_v[0, pl.ds(j * 16, 16)]
      for t in range(1, NS):
        s = s + red_v[t, pl.ds(j * 16, 16)]
      cnt_red[pl.ds(j * 16, 16)] = s
      return carry

    lax.fori_loop(0, rows_ps // 16, red_body, 0)

    pltpu.sync_copy(acc_sh.at[pl.ds(r0, rows_ps)],
                    acc_out.at[cid].at[pl.ds(r0, rows_ps)])
    pltpu.sync_copy(cnt_red, cnt_out.at[cid].at[pl.ds(r0, rows_ps)])

  return seg_sum


N1_PAD = 6144
N2_PAD = 2048
L1_CHUNKS = 40
L2_CHUNKS = 16

_seg_sum_l1 = _make_seg_sum(L1_CHUNKS, N1_PAD)
_seg_sum_l2 = _make_seg_sum(L2_CHUNKS, N2_PAD)


BLK = 1000


def _prompt_body(x_ref, m_ref, wpin_ref, bpin_ref, wpout_ref, bpout_ref,
                 wn1_ref, ws1_ref, b1_ref, g0_ref, s1_ref):
  x = x_ref[...]
  pin = jnp.maximum(
      jnp.dot(x, wpin_ref[...], preferred_element_type=jnp.float32)
      + bpin_ref[...], 0.0)
  pout = jnp.maximum(
      jnp.dot(x, wpout_ref[...], preferred_element_type=jnp.float32)
      + bpout_ref[...], 0.0)
  s = jnp.where(m_ref[...] > 0, pin, pout)
  g0_ref[...] = (
      jnp.dot(x, wn1_ref[0:IN_DIM], preferred_element_type=jnp.float32)
      + jnp.dot(s, wn1_ref[IN_DIM:IN_DIM + PROMPT],
                preferred_element_type=jnp.float32))
  s1_ref[...] = (
      jnp.dot(x, ws1_ref[0:IN_DIM], preferred_element_type=jnp.float32)
      + jnp.dot(s, ws1_ref[IN_DIM:IN_DIM + PROMPT],
                preferred_element_type=jnp.float32)
      + b1_ref[...])


def _layer1_body(acc_ref, cnt_ref, s1_ref, wn2_ref, ws2_ref, b2_ref,
                 g1_ref, s2_ref):
  a = acc_ref[0] + acc_ref[1]
  d = cnt_ref[0] + cnt_ref[1]
  h1 = jnp.maximum(s1_ref[...] + a / jnp.maximum(d, 1.0), 0.0)
  g1_ref[...] = jnp.dot(h1, wn2_ref[...], preferred_element_type=jnp.float32)
  s2_ref[...] = (jnp.dot(h1, ws2_ref[...], preferred_element_type=jnp.float32)
                 + b2_ref[...])


def _layer2_body(acc_ref, cnt_ref, s2_ref, wc_ref, bc_ref, out_ref):
  a = acc_ref[0] + acc_ref[1]
  d = cnt_ref[0] + cnt_ref[1]
  h2 = s2_ref[...] + a / jnp.maximum(d, 1.0)
  out_ref[...] = (jnp.dot(h2, wc_ref[...], preferred_element_type=jnp.float32)
                  + bc_ref[...])


def _full(shape):
  return pl.BlockSpec(shape, lambda i: (0,) * len(shape))


def kernel(features, membership_mask, block0_src, block0_dst, block1_src,
           block1_dst, output_nodes_indices, W_pin, b_pin, W_pout, b_pout,
           Ws1, Wn1, b1, Ws2, Wn2, b2, Wc, bc):
  del output_nodes_indices
  f32 = jnp.float32
  mask_i = membership_mask.astype(jnp.int32).reshape(N0, 1)

  g0, s1 = pl.pallas_call(
      _prompt_body,
      grid=(N0 // BLK,),
      in_specs=[
          pl.BlockSpec((BLK, IN_DIM), lambda i: (i, 0)),
          pl.BlockSpec((BLK, 1), lambda i: (i, 0)),
          _full((IN_DIM, PROMPT)),
          _full((1, PROMPT)),
          _full((IN_DIM, PROMPT)),
          _full((1, PROMPT)),
          _full((IN_DIM + PROMPT, HID)),
          _full((IN_DIM + PROMPT, HID)),
          _full((1, HID)),
      ],
      out_specs=[
          pl.BlockSpec((BLK, HID), lambda i: (i, 0)),
          pl.BlockSpec((BLK, HID), lambda i: (i, 0)),
      ],
      out_shape=[
          jax.ShapeDtypeStruct((N0, HID), f32),
          jax.ShapeDtypeStruct((N0, HID), f32),
      ],
  )(features, mask_i, W_pin, b_pin.reshape(1, PROMPT), W_pout,
    b_pout.reshape(1, PROMPT), Wn1, Ws1, b1.reshape(1, HID))

  e0_pad = NW * L1_CHUNKS * CHUNK
  src0 = jnp.concatenate([block0_src, jnp.zeros((e0_pad - E0,), jnp.int32)])
  dst0 = jnp.concatenate([block0_dst, jnp.full((e0_pad - E0,), N1, jnp.int32)])
  zrows1 = jnp.zeros((N1_PAD // NS, HID), f32)
  zcnt1 = jnp.zeros((N1_PAD,), f32)
  acc1, cnt1 = _seg_sum_l1(g0, src0, dst0, zrows1, zcnt1)
  cnt1 = cnt1.reshape(NC, N1_PAD, 1)

  g1, s2 = pl.pallas_call(
      _layer1_body,
      grid=(N1 // BLK,),
      in_specs=[
          pl.BlockSpec((NC, BLK, HID), lambda i: (0, i, 0)),
          pl.BlockSpec((NC, BLK, 1), lambda i: (0, i, 0)),
          pl.BlockSpec((BLK, HID), lambda i: (i, 0)),
          _full((HID, HID)),
          _full((HID, HID)),
          _full((1, HID)),
      ],
      out_specs=[
          pl.BlockSpec((BLK, HID), lambda i: (i, 0)),
          pl.BlockSpec((BLK, HID), lambda i: (i, 0)),
      ],
      out_shape=[
          jax.ShapeDtypeStruct((N1, HID), f32),
          jax.ShapeDtypeStruct((N1, HID), f32),
      ],
  )(acc1, cnt1, s1, Wn2, Ws2, b2.reshape(1, HID))

  e1_pad = NW * L2_CHUNKS * CHUNK
  src1 = jnp.concatenate([block1_src, jnp.zeros((e1_pad - E1,), jnp.int32)])
  dst1 = jnp.concatenate([block1_dst, jnp.full((e1_pad - E1,), N2, jnp.int32)])
  zrows2 = jnp.zeros((N2_PAD // NS, HID), f32)
  zcnt2 = jnp.zeros((N2_PAD,), f32)
  acc2, cnt2 = _seg_sum_l2(g1, src1, dst1, zrows2, zcnt2)
  cnt2 = cnt2.reshape(NC, N2_PAD, 1)

  logits = pl.pallas_call(
      _layer2_body,
      grid=(N2 // BLK,),
      in_specs=[
          pl.BlockSpec((NC, BLK, HID), lambda i: (0, i, 0)),
          pl.BlockSpec((NC, BLK, 1), lambda i: (0, i, 0)),
          pl.BlockSpec((BLK, HID), lambda i: (i, 0)),
          _full((HID, OUT)),
          _full((1, OUT)),
      ],
      out_specs=pl.BlockSpec((BLK, OUT), lambda i: (i, 0)),
      out_shape=jax.ShapeDtypeStruct((N2, OUT), f32),
  )(acc2, cnt2, s2, Wc, bc.reshape(1, OUT))

  return logits


# speedup vs baseline: 4.5981x; 1.0957x over previous
import functools

import jax
import jax.numpy as jnp
from jax import lax
from jax.experimental import pallas as pl
from jax.experimental.pallas import tpu as pltpu
from jax.experimental.pallas import tpu_sc as plsc

N0, N1, N2 = 10000, 5000, 2000
E0, E1 = 160000, 64000
IN_DIM, PROMPT, HID, OUT = 128, 64, 128, 64

NC, NS = 2, 16
NW = NC * NS
CHUNK = 128
NBUF = 2


def _make_seg_sum(n_chunks: int, npad: int):
  e_pw = n_chunks * CHUNK
  rows_ps = npad // NS
  assert npad % NS == 0

  mesh = plsc.VectorSubcoreMesh(core_axis_name="c", subcore_axis_name="s")

  @functools.partial(
      pl.kernel,
      mesh=mesh,
      compiler_params=pltpu.CompilerParams(needs_layout_passes=False),
      out_type=[
          jax.ShapeDtypeStruct((NC, npad, HID), jnp.float32),
          jax.ShapeDtypeStruct((NC, npad), jnp.float32),
      ],
      scratch_types=[
          pltpu.VMEM((NBUF, CHUNK), jnp.int32),
          pltpu.VMEM((NBUF, CHUNK), jnp.int32),
          [pltpu.VMEM((CHUNK, HID), jnp.float32)] * NBUF,
          pltpu.VMEM((npad,), jnp.float32),
          pltpu.VMEM((NS, rows_ps), jnp.float32),
          pltpu.VMEM((rows_ps,), jnp.float32),
          pltpu.VMEM_SHARED((npad, HID), jnp.float32),
          pltpu.VMEM_SHARED((NS, npad), jnp.float32),
          [pltpu.SemaphoreType.DMA] * NBUF,
      ],
  )
  def seg_sum(table, src, dst, zrows, zcnt, acc_out, cnt_out,
              src_v, dst_v, rows_v, cnt_loc, red_v, cnt_red, acc_sh,
              cnt_stage, sems):
    cid = lax.axis_index("c")
    sid = lax.axis_index("s")
    wid = sid * NC + cid

    r0 = sid * rows_ps
    pltpu.sync_copy(zrows, acc_sh.at[pl.ds(r0, rows_ps)])
    pltpu.sync_copy(zcnt, cnt_loc)
    plsc.subcore_barrier()

    base = wid * e_pw
    ones16 = jnp.ones((16,), jnp.float32)

    def load_and_fire(ci, b):
      off = base + ci * CHUNK
      pltpu.sync_copy(src.at[pl.ds(off, CHUNK)], src_v.at[b])
      pltpu.sync_copy(dst.at[pl.ds(off, CHUNK)], dst_v.at[b])
      pltpu.async_copy(table.at[src_v.at[b]], rows_v[b], sems[b])

    def drain(b):
      pltpu.make_async_copy(table.at[src_v.at[b]], rows_v[b], sems[b]).wait()
      pltpu.sync_copy(rows_v[b], acc_sh.at[dst_v.at[b]], add=True)
      for j in range(CHUNK // 16):
        idx = dst_v[b, pl.ds(j * 16, 16)]
        plsc.addupdate_scatter(cnt_loc, [idx], ones16)

    load_and_fire(0, 0)
    n2 = n_chunks // 2

    def body(i2, carry):
      i = 2 * i2
      load_and_fire(i + 1, 1)
      drain(0)

      @pl.when(i2 + 1 < n2)
      def _prefetch():
        load_and_fire(i + 2, 0)

      drain(1)
      return carry

    lax.fori_loop(0, n2, body, 0)
    pltpu.sync_copy(cnt_loc, cnt_stage.at[sid])
    plsc.subcore_barrier()

    pltpu.sync_copy(cnt_stage.at[:, pl.ds(r0, rows_ps)], red_v)

    def red_body(j, carry):
      s = red_v[0, pl.ds(j * 16, 16)]
      for t in range(1, NS):
        s = s + red_v[t, pl.ds(j * 16, 16)]
      cnt_red[pl.ds(j * 16, 16)] = s
      return carry

    lax.fori_loop(0, rows_ps // 16, red_body, 0)

    pltpu.sync_copy(acc_sh.at[pl.ds(r0, rows_ps)],
                    acc_out.at[cid].at[pl.ds(r0, rows_ps)])
    pltpu.sync_copy(cnt_red, cnt_out.at[cid].at[pl.ds(r0, rows_ps)])

  return seg_sum


N1_PAD = 6144
N2_PAD = 2048
L1_CHUNKS = 40
L2_CHUNKS = 16

_seg_sum_l1 = _make_seg_sum(L1_CHUNKS, N1_PAD)
_seg_sum_l2 = _make_seg_sum(L2_CHUNKS, N2_PAD)


BLK = 1000


def _prompt_body(x_ref, m_ref, wpin_ref, bpin_ref, wpout_ref, bpout_ref,
                 wn1_ref, ws1_ref, b1_ref, g0_ref, s1_ref):
  x = x_ref[...]
  pin = jnp.maximum(
      jnp.dot(x, wpin_ref[...], preferred_element_type=jnp.float32)
      + bpin_ref[...], 0.0)
  pout = jnp.maximum(
      jnp.dot(x, wpout_ref[...], preferred_element_type=jnp.float32)
      + bpout_ref[...], 0.0)
  s = jnp.where(m_ref[...] > 0, pin, pout)
  g0_ref[...] = (
      jnp.dot(x, wn1_ref[0:IN_DIM], preferred_element_type=jnp.float32)
      + jnp.dot(s, wn1_ref[IN_DIM:IN_DIM + PROMPT],
                preferred_element_type=jnp.float32))
  s1_ref[...] = (
      jnp.dot(x, ws1_ref[0:IN_DIM], preferred_element_type=jnp.float32)
      + jnp.dot(s, ws1_ref[IN_DIM:IN_DIM + PROMPT],
                preferred_element_type=jnp.float32)
      + b1_ref[...])


def _layer1_body(acc_ref, cnt_ref, s1_ref, wn2_ref, ws2_ref, b2_ref,
                 g1_ref, s2_ref):
  a = acc_ref[0] + acc_ref[1]
  d = cnt_ref[0] + cnt_ref[1]
  h1 = jnp.maximum(s1_ref[...] + a / jnp.maximum(d, 1.0), 0.0)
  g1_ref[...] = jnp.dot(h1, wn2_ref[...], preferred_element_type=jnp.float32)
  s2_ref[...] = (jnp.dot(h1, ws2_ref[...], preferred_element_type=jnp.float32)
                 + b2_ref[...])


def _layer2_body(acc_ref, cnt_ref, s2_ref, wc_ref, bc_ref, out_ref):
  a = acc_ref[0] + acc_ref[1]
  d = cnt_ref[0] + cnt_ref[1]
  h2 = s2_ref[...] + a / jnp.maximum(d, 1.0)
  out_ref[...] = (jnp.dot(h2, wc_ref[...], preferred_element_type=jnp.float32)
                  + bc_ref[...])


def _full(shape):
  return pl.BlockSpec(shape, lambda i: (0,) * len(shape))


def kernel(features, membership_mask, block0_src, block0_dst, block1_src,
           block1_dst, output_nodes_indices, W_pin, b_pin, W_pout, b_pout,
           Ws1, Wn1, b1, Ws2, Wn2, b2, Wc, bc):
  del output_nodes_indices
  f32 = jnp.float32
  mask_i = membership_mask.astype(jnp.int32).reshape(N0, 1)

  g0, s1 = pl.pallas_call(
      _prompt_body,
      grid=(N0 // BLK,),
      in_specs=[
          pl.BlockSpec((BLK, IN_DIM), lambda i: (i, 0)),
          pl.BlockSpec((BLK, 1), lambda i: (i, 0)),
          _full((IN_DIM, PROMPT)),
          _full((1, PROMPT)),
          _full((IN_DIM, PROMPT)),
          _full((1, PROMPT)),
          _full((IN_DIM + PROMPT, HID)),
          _full((IN_DIM + PROMPT, HID)),
          _full((1, HID)),
      ],
      out_specs=[
          pl.BlockSpec((BLK, HID), lambda i: (i, 0)),
          pl.BlockSpec((BLK, HID), lambda i: (i, 0)),
      ],
      out_shape=[
          jax.ShapeDtypeStruct((N0, HID), f32),
          jax.ShapeDtypeStruct((N0, HID), f32),
      ],
  )(features, mask_i, W_pin, b_pin.reshape(1, PROMPT), W_pout,
    b_pout.reshape(1, PROMPT), Wn1, Ws1, b1.reshape(1, HID))

  e0_pad = NW * L1_CHUNKS * CHUNK
  src0 = jnp.concatenate([block0_src, jnp.zeros((e0_pad - E0,), jnp.int32)])
  dst0 = jnp.concatenate([block0_dst, jnp.full((e0_pad - E0,), N1, jnp.int32)])
  zrows1 = jnp.zeros((N1_PAD // NS, HID), f32)
  zcnt1 = jnp.zeros((N1_PAD,), f32)
  acc1, cnt1 = _seg_sum_l1(g0, src0, dst0, zrows1, zcnt1)
  cnt1 = cnt1.reshape(NC, N1_PAD, 1)

  g1, s2 = pl.pallas_call(
      _layer1_body,
      grid=(N1 // BLK,),
      in_specs=[
          pl.BlockSpec((NC, BLK, HID), lambda i: (0, i, 0)),
          pl.BlockSpec((NC, BLK, 1), lambda i: (0, i, 0)),
          pl.BlockSpec((BLK, HID), lambda i: (i, 0)),
          _full((HID, HID)),
          _full((HID, HID)),
          _full((1, HID)),
      ],
      out_specs=[
          pl.BlockSpec((BLK, HID), lambda i: (i, 0)),
          pl.BlockSpec((BLK, HID), lambda i: (i, 0)),
      ],
      out_shape=[
          jax.ShapeDtypeStruct((N1, HID), f32),
          jax.ShapeDtypeStruct((N1, HID), f32),
      ],
  )(acc1, cnt1, s1, Wn2, Ws2, b2.reshape(1, HID))

  e1_pad = NW * L2_CHUNKS * CHUNK
  src1 = jnp.concatenate([block1_src, jnp.zeros((e1_pad - E1,), jnp.int32)])
  dst1 = jnp.concatenate([block1_dst, jnp.full((e1_pad - E1,), N2, jnp.int32)])
  zrows2 = jnp.zeros((N2_PAD // NS, HID), f32)
  zcnt2 = jnp.zeros((N2_PAD,), f32)
  acc2, cnt2 = _seg_sum_l2(g1, src1, dst1, zrows2, zcnt2)
  cnt2 = cnt2.reshape(NC, N2_PAD, 1)

  logits = pl.pallas_call(
      _layer2_body,
      grid=(N2 // BLK,),
      in_specs=[
          pl.BlockSpec((NC, BLK, HID), lambda i: (0, i, 0)),
          pl.BlockSpec((NC, BLK, 1), lambda i: (0, i, 0)),
          pl.BlockSpec((BLK, HID), lambda i: (i, 0)),
          _full((HID, OUT)),
          _full((1, OUT)),
      ],
      out_specs=pl.BlockSpec((BLK, OUT), lambda i: (i, 0)),
      out_shape=jax.ShapeDtypeStruct((N2, OUT), f32),
  )(acc2, cnt2, s2, Wc, bc.reshape(1, OUT))

  return logits
